# baseline (speedup 1.0000x reference)
import functools

import jax
import jax.numpy as jnp
from jax import lax
from jax.experimental import pallas as pl
from jax.experimental.pallas import tpu as pltpu
from jax.experimental.pallas import tpu_sc as plsc

_B = 16384
_D = 64
_U = 100000
_NC = 2
_NS = 16
_NW = _NC * _NS
_CPT = _D // _NW
_OCH = 2048


def _sc_gather_cols(uidx, iidx, tu_mlp, tu_mf, tu_v, tu_at, ti_mlp, ti_mf):
    mesh = plsc.VectorSubcoreMesh(core_axis_name="c", subcore_axis_name="s")
    out_sds = [jax.ShapeDtypeStruct((_D, _B), jnp.float32)] * 6

    @functools.partial(
        pl.kernel,
        mesh=mesh,
        out_type=out_sds,
        compiler_params=pltpu.CompilerParams(needs_layout_passes=False),
        scratch_types=[
            pltpu.VMEM((_B,), jnp.int32),
            pltpu.VMEM((_U,), jnp.float32),
            pltpu.VMEM((_OCH,), jnp.float32),
            pltpu.VMEM((_OCH,), jnp.float32),
            pltpu.SemaphoreType.DMA,
            pltpu.SemaphoreType.DMA,
        ],
    )
    def k(uidx_hbm, iidx_hbm, umlp, umf, uv, uat, imlp, imf,
          o_umlp, o_umf, o_uv, o_uat, o_imlp, o_imf,
          idx_v, col_v, out_v0, out_v1, sem0, sem1):
        wid = lax.axis_index("s") * _NC + lax.axis_index("c")
        obufs = ((out_v0, sem0), (out_v1, sem1))

        def gather_column(tbl, out, f):
            pltpu.sync_copy(tbl.at[f], col_v)

            def pair(it, _):
                for b in range(2):
                    c = it * 2 + b
                    ov, sem = obufs[b]

                    @pl.when(it > 0)
                    def _():
                        pltpu.make_async_copy(
                            ov, out.at[f, pl.ds(0, _OCH)], sem).wait()

                    def grp(g, _):
                        base = g * _L4
                        for u in range(_L4 // 16):
                            idxv = idx_v[pl.ds(c * _OCH + base + u * 16, 16)]
                            ov[pl.ds(base + u * 16, 16)] = (
                                plsc.load_gather(col_v, [idxv]))
                        return 0
                    lax.fori_loop(0, _OCH // _L4, grp, 0, unroll=False)
                    pltpu.async_copy(ov, out.at[f, pl.ds(c * _OCH, _OCH)], sem)
                return 0
            lax.fori_loop(0, _B // (2 * _OCH), pair, 0, unroll=False)
            for b in range(2):
                ov, sem = obufs[b]
                pltpu.make_async_copy(
                    ov, out.at[f, pl.ds(0, _OCH)], sem).wait()

        _L4 = 256

        pltpu.sync_copy(uidx_hbm, idx_v)
        for tbl, out in ((umlp, o_umlp), (umf, o_umf), (uv, o_uv),
                         (uat, o_uat)):
            for kk in range(_CPT):
                gather_column(tbl, out, wid * _CPT + kk)
        pltpu.sync_copy(iidx_hbm, idx_v)
        for tbl, out in ((imlp, o_imlp), (imf, o_imf)):
            for kk in range(_CPT):
                gather_column(tbl, out, wid * _CPT + kk)

    return k(uidx, iidx, tu_mlp, tu_mf, tu_v, tu_at, ti_mlp, ti_mf)


def _dgT(w, x):
    return lax.dot_general(w, x, (((0,), (0,)), ((), ())),
                           preferred_element_type=jnp.float32)


def _tc_poster_body(poster_ref, feW0_ref, feb0_ref, feW1_ref, feb1_ref,
                    pe_ref):
    f32 = jnp.float32
    h = jnp.maximum(
        lax.dot_general(feW0_ref[...], poster_ref[...],
                        (((0,), (1,)), ((), ())),
                        preferred_element_type=f32) + feb0_ref[...], 0.0)
    pe_ref[...] = _dgT(feW1_ref[...], h) + feb1_ref[...]


def _tc_poster(poster, feW0, feb0, feW1, feb1, bm=1024):
    return pl.pallas_call(
        _tc_poster_body,
        grid=(_B // bm,),
        in_specs=[
            pl.BlockSpec((bm, 2048), lambda i: (i, 0)),
            pl.BlockSpec(feW0.shape, lambda i: (0, 0)),
            pl.BlockSpec(feb0.shape, lambda i: (0, 0)),
            pl.BlockSpec(feW1.shape, lambda i: (0, 0)),
            pl.BlockSpec(feb1.shape, lambda i: (0, 0)),
        ],
        out_specs=pl.BlockSpec((_D, bm), lambda i: (0, i)),
        out_shape=jax.ShapeDtypeStruct((_D, _B), jnp.float32),
    )(poster, feW0, feb0, feW1, feb1)


def _tc_body(pe_ref, guml_ref, gumf_ref, guv_ref, guat_ref,
             giml_ref, gimf_ref,
             fcW0_ref, fcb0_ref, fcW1_ref, fcb1_ref,
             fvW0_ref, fvb0_ref, fvW1_ref, fvb1_ref,
             atW_ref, atb_ref, afW_ref, afb_ref, out_ref):
    f32 = jnp.float32
    pe_t = pe_ref[...]
    mlp_in = jnp.concatenate([guml_ref[...], giml_ref[...]], axis=0)
    mlp = jnp.maximum(_dgT(fcW0_ref[...], mlp_in) + fcb0_ref[...], 0.0)
    mlp = jnp.maximum(_dgT(fcW1_ref[...], mlp) + fcb1_ref[...], 0.0)

    v_in = jnp.concatenate([guv_ref[...], pe_t], axis=0)
    vv = jnp.maximum(_dgT(fvW0_ref[...], v_in) + fvb0_ref[...], 0.0)
    vv = jnp.maximum(_dgT(fvW1_ref[...], vv) + fvb1_ref[...], 0.0)

    att = jax.nn.sigmoid(
        _dgT(atW_ref[...], jnp.maximum(guat_ref[...], 0.0)) + atb_ref[...])

    mf = gumf_ref[...] * gimf_ref[...]
    vec = jnp.concatenate(
        [mlp * att[0:1, :], mf * att[1:2, :], vv * att[2:3, :]], axis=0)
    out_ref[...] = jax.nn.sigmoid(_dgT(afW_ref[...], vec) + afb_ref[...])


def _tc_dense(pe, guml, gumf, guv, guat, giml, gimf,
              fcW0, fcb0, fcW1, fcb1,
              fvW0, fvb0, fvW1, fvb1, atW, atb, afW, afb, bm=1024):
    grid = (_B // bm,)

    def col_spec(rows):
        return pl.BlockSpec((rows, bm), lambda i: (0, i))

    def full_spec(shape):
        return pl.BlockSpec(shape, lambda i: tuple(0 for _ in shape))

    in_specs = [
        col_spec(_D),
        col_spec(_D), col_spec(_D), col_spec(_D),
        col_spec(_D), col_spec(_D), col_spec(_D),
        full_spec(fcW0.shape), full_spec(fcb0.shape),
        full_spec(fcW1.shape), full_spec(fcb1.shape),
        full_spec(fvW0.shape), full_spec(fvb0.shape),
        full_spec(fvW1.shape), full_spec(fvb1.shape),
        full_spec(atW.shape), full_spec(atb.shape),
        full_spec(afW.shape), full_spec(afb.shape),
    ]
    return pl.pallas_call(
        _tc_body,
        grid=grid,
        in_specs=in_specs,
        out_specs=pl.BlockSpec((1, bm), lambda i: (0, i)),
        out_shape=jax.ShapeDtypeStruct((1, _B), jnp.float32),
    )(pe, guml, gumf, guv, guat, giml, gimf,
      fcW0, fcb0, fcW1, fcb1,
      fvW0, fvb0, fvW1, fvb1, atW, atb, afW, afb)


def kernel(user_indices, item_indices, poster_embeddings,
           emb_user_mlp, emb_item_mlp, emb_user_mf, emb_item_mf,
           emb_user_v, emb_atten,
           fe_W0, fe_b0, fe_W1, fe_b1, fc_W0, fc_b0, fc_W1, fc_b1,
           fv_W0, fv_b0, fv_W1, fv_b1, at_W, at_b, af_W, af_b):
    guml, gumf, guv, guat, giml, gimf = _sc_gather_cols(
        user_indices, item_indices,
        emb_user_mlp.T, emb_user_mf.T, emb_user_v.T, emb_atten.T,
        emb_item_mlp.T, emb_item_mf.T)
    pe = _tc_poster(poster_embeddings, fe_W0, fe_b0.reshape(-1, 1),
                    fe_W1, fe_b1.reshape(-1, 1))
    out_t = _tc_dense(
        pe, guml, gumf, guv, guat, giml, gimf,
        fc_W0, fc_b0.reshape(-1, 1), fc_W1, fc_b1.reshape(-1, 1),
        fv_W0, fv_b0.reshape(-1, 1), fv_W1, fv_b1.reshape(-1, 1),
        at_W, at_b.reshape(-1, 1), af_W, af_b.reshape(-1, 1))
    return out_t.reshape(_B, 1)


# speedup vs baseline: 1.0334x; 1.0334x over previous
import functools

import jax
import jax.numpy as jnp
from jax import lax
from jax.experimental import pallas as pl
from jax.experimental.pallas import tpu as pltpu
from jax.experimental.pallas import tpu_sc as plsc

_B = 16384
_D = 64
_U = 100000
_NC = 2
_NS = 16
_NW = _NC * _NS
_CPT = _D // _NW
_OCH = 2048


def _sc_gather_cols(uidx, iidx, tu_mlp, tu_mf, tu_v, tu_at, ti_mlp, ti_mf):
    mesh = plsc.VectorSubcoreMesh(core_axis_name="c", subcore_axis_name="s")
    out_sds = [jax.ShapeDtypeStruct((_D, _B), jnp.float32)] * 6

    @functools.partial(
        pl.kernel,
        mesh=mesh,
        out_type=out_sds,
        compiler_params=pltpu.CompilerParams(needs_layout_passes=False),
        scratch_types=[
            pltpu.VMEM((_B,), jnp.int32),
            pltpu.VMEM((_U,), jnp.float32),
            pltpu.VMEM((_OCH,), jnp.float32),
            pltpu.VMEM((_OCH,), jnp.float32),
            pltpu.SemaphoreType.DMA,
            pltpu.SemaphoreType.DMA,
        ],
    )
    def k(uidx_hbm, iidx_hbm, umlp, umf, uv, uat, imlp, imf,
          o_umlp, o_umf, o_uv, o_uat, o_imlp, o_imf,
          idx_v, col_v, out_v0, out_v1, sem0, sem1):
        wid = lax.axis_index("s") * _NC + lax.axis_index("c")
        obufs = ((out_v0, sem0), (out_v1, sem1))

        def gather_column(tbl, out, f):
            pltpu.sync_copy(tbl.at[f], col_v)

            def pair(it, _):
                for b in range(2):
                    c = it * 2 + b
                    ov, sem = obufs[b]

                    @pl.when(it > 0)
                    def _():
                        pltpu.make_async_copy(
                            ov, out.at[f, pl.ds(0, _OCH)], sem).wait()

                    def grp(g, _):
                        base = g * _L4
                        for u in range(_L4 // 16):
                            idxv = idx_v[pl.ds(c * _OCH + base + u * 16, 16)]
                            ov[pl.ds(base + u * 16, 16)] = (
                                plsc.load_gather(col_v, [idxv]))
                        return 0
                    lax.fori_loop(0, _OCH // _L4, grp, 0, unroll=False)
                    pltpu.async_copy(ov, out.at[f, pl.ds(c * _OCH, _OCH)], sem)
                return 0
            lax.fori_loop(0, _B // (2 * _OCH), pair, 0, unroll=False)
            for b in range(2):
                ov, sem = obufs[b]
                pltpu.make_async_copy(
                    ov, out.at[f, pl.ds(0, _OCH)], sem).wait()

        _L4 = 128

        pltpu.sync_copy(uidx_hbm, idx_v)
        for tbl, out in ((umlp, o_umlp), (umf, o_umf), (uv, o_uv),
                         (uat, o_uat)):
            for kk in range(_CPT):
                gather_column(tbl, out, wid * _CPT + kk)
        pltpu.sync_copy(iidx_hbm, idx_v)
        for tbl, out in ((imlp, o_imlp), (imf, o_imf)):
            for kk in range(_CPT):
                gather_column(tbl, out, wid * _CPT + kk)

    return k(uidx, iidx, tu_mlp, tu_mf, tu_v, tu_at, ti_mlp, ti_mf)


def _dgT(w, x):
    return lax.dot_general(w, x, (((0,), (0,)), ((), ())),
                           preferred_element_type=jnp.float32)


def _tc_poster_body(poster_ref, feW0_ref, feb0_ref, feW1_ref, feb1_ref,
                    pe_ref):
    f32 = jnp.float32
    h = jnp.maximum(
        lax.dot_general(feW0_ref[...], poster_ref[...],
                        (((0,), (1,)), ((), ())),
                        preferred_element_type=f32) + feb0_ref[...], 0.0)
    pe_ref[...] = _dgT(feW1_ref[...], h) + feb1_ref[...]


def _tc_poster(poster, feW0, feb0, feW1, feb1, bm=1024):
    return pl.pallas_call(
        _tc_poster_body,
        grid=(_B // bm,),
        in_specs=[
            pl.BlockSpec((bm, 2048), lambda i: (i, 0)),
            pl.BlockSpec(feW0.shape, lambda i: (0, 0)),
            pl.BlockSpec(feb0.shape, lambda i: (0, 0)),
            pl.BlockSpec(feW1.shape, lambda i: (0, 0)),
            pl.BlockSpec(feb1.shape, lambda i: (0, 0)),
        ],
        out_specs=pl.BlockSpec((_D, bm), lambda i: (0, i)),
        out_shape=jax.ShapeDtypeStruct((_D, _B), jnp.float32),
    )(poster, feW0, feb0, feW1, feb1)


def _tc_body(pe_ref, guml_ref, gumf_ref, guv_ref, guat_ref,
             giml_ref, gimf_ref,
             fcW0_ref, fcb0_ref, fcW1_ref, fcb1_ref,
             fvW0_ref, fvb0_ref, fvW1_ref, fvb1_ref,
             atW_ref, atb_ref, afW_ref, afb_ref, out_ref):
    f32 = jnp.float32
    pe_t = pe_ref[...]
    mlp_in = jnp.concatenate([guml_ref[...], giml_ref[...]], axis=0)
    mlp = jnp.maximum(_dgT(fcW0_ref[...], mlp_in) + fcb0_ref[...], 0.0)
    mlp = jnp.maximum(_dgT(fcW1_ref[...], mlp) + fcb1_ref[...], 0.0)

    v_in = jnp.concatenate([guv_ref[...], pe_t], axis=0)
    vv = jnp.maximum(_dgT(fvW0_ref[...], v_in) + fvb0_ref[...], 0.0)
    vv = jnp.maximum(_dgT(fvW1_ref[...], vv) + fvb1_ref[...], 0.0)

    att = jax.nn.sigmoid(
        _dgT(atW_ref[...], jnp.maximum(guat_ref[...], 0.0)) + atb_ref[...])

    mf = gumf_ref[...] * gimf_ref[...]
    vec = jnp.concatenate(
        [mlp * att[0:1, :], mf * att[1:2, :], vv * att[2:3, :]], axis=0)
    out_ref[...] = jax.nn.sigmoid(_dgT(afW_ref[...], vec) + afb_ref[...])


def _tc_dense(pe, guml, gumf, guv, guat, giml, gimf,
              fcW0, fcb0, fcW1, fcb1,
              fvW0, fvb0, fvW1, fvb1, atW, atb, afW, afb, bm=2048):
    grid = (_B // bm,)

    def col_spec(rows):
        return pl.BlockSpec((rows, bm), lambda i: (0, i))

    def full_spec(shape):
        return pl.BlockSpec(shape, lambda i: tuple(0 for _ in shape))

    in_specs = [
        col_spec(_D),
        col_spec(_D), col_spec(_D), col_spec(_D),
        col_spec(_D), col_spec(_D), col_spec(_D),
        full_spec(fcW0.shape), full_spec(fcb0.shape),
        full_spec(fcW1.shape), full_spec(fcb1.shape),
        full_spec(fvW0.shape), full_spec(fvb0.shape),
        full_spec(fvW1.shape), full_spec(fvb1.shape),
        full_spec(atW.shape), full_spec(atb.shape),
        full_spec(afW.shape), full_spec(afb.shape),
    ]
    return pl.pallas_call(
        _tc_body,
        grid=grid,
        in_specs=in_specs,
        out_specs=pl.BlockSpec((1, bm), lambda i: (0, i)),
        out_shape=jax.ShapeDtypeStruct((1, _B), jnp.float32),
    )(pe, guml, gumf, guv, guat, giml, gimf,
      fcW0, fcb0, fcW1, fcb1,
      fvW0, fvb0, fvW1, fvb1, atW, atb, afW, afb)


def kernel(user_indices, item_indices, poster_embeddings,
           emb_user_mlp, emb_item_mlp, emb_user_mf, emb_item_mf,
           emb_user_v, emb_atten,
           fe_W0, fe_b0, fe_W1, fe_b1, fc_W0, fc_b0, fc_W1, fc_b1,
           fv_W0, fv_b0, fv_W1, fv_b1, at_W, at_b, af_W, af_b):
    guml, gumf, guv, guat, giml, gimf = _sc_gather_cols(
        user_indices, item_indices,
        emb_user_mlp.T, emb_user_mf.T, emb_user_v.T, emb_atten.T,
        emb_item_mlp.T, emb_item_mf.T)
    pe = _tc_poster(poster_embeddings, fe_W0, fe_b0.reshape(-1, 1),
                    fe_W1, fe_b1.reshape(-1, 1))
    out_t = _tc_dense(
        pe, guml, gumf, guv, guat, giml, gimf,
        fc_W0, fc_b0.reshape(-1, 1), fc_W1, fc_b1.reshape(-1, 1),
        fv_W0, fv_b0.reshape(-1, 1), fv_W1, fv_b1.reshape(-1, 1),
        at_W, at_b.reshape(-1, 1), af_W, af_b.reshape(-1, 1))
    return out_t.reshape(_B, 1)


# speedup vs baseline: 1.0373x; 1.0038x over previous
import functools

import jax
import jax.numpy as jnp
from jax import lax
from jax.experimental import pallas as pl
from jax.experimental.pallas import tpu as pltpu
from jax.experimental.pallas import tpu_sc as plsc

_B = 16384
_D = 64
_U = 100000
_NC = 2
_NS = 16
_NW = _NC * _NS
_CPT = _D // _NW
_OCH = 2048


def _sc_gather_cols(uidx, iidx, tu_mlp, tu_mf, tu_v, tu_at, ti_mlp, ti_mf):
    mesh = plsc.VectorSubcoreMesh(core_axis_name="c", subcore_axis_name="s")
    out_sds = [jax.ShapeDtypeStruct((_D, _B), jnp.float32)] * 6

    @functools.partial(
        pl.kernel,
        mesh=mesh,
        out_type=out_sds,
        compiler_params=pltpu.CompilerParams(needs_layout_passes=False),
        scratch_types=[
            pltpu.VMEM((_B,), jnp.int32),
            pltpu.VMEM((_U,), jnp.float32),
            pltpu.VMEM((_OCH,), jnp.float32),
            pltpu.VMEM((_OCH,), jnp.float32),
            pltpu.SemaphoreType.DMA,
            pltpu.SemaphoreType.DMA,
            pltpu.SemaphoreType.DMA,
        ],
    )
    def k(uidx_hbm, iidx_hbm, umlp, umf, uv, uat, imlp, imf,
          o_umlp, o_umf, o_uv, o_uat, o_imlp, o_imf,
          idx_v, col_v, out_v0, out_v1, sem0, sem1, sem_c):
        wid = lax.axis_index("s") * _NC + lax.axis_index("c")
        obufs = ((out_v0, sem0), (out_v1, sem1))

        def gather_column(tbl, out, f):
            def pair(it, _):
                for b in range(2):
                    c = it * 2 + b
                    ov, sem = obufs[b]

                    @pl.when(it > 0)
                    def _():
                        pltpu.make_async_copy(
                            ov, out.at[f, pl.ds(0, _OCH)], sem).wait()

                    def grp(g, _):
                        base = g * _L4
                        for u in range(_L4 // 16):
                            idxv = idx_v[pl.ds(c * _OCH + base + u * 16, 16)]
                            ov[pl.ds(base + u * 16, 16)] = (
                                plsc.load_gather(col_v, [idxv]))
                        return 0
                    lax.fori_loop(0, _OCH // _L4, grp, 0, unroll=False)
                    pltpu.async_copy(ov, out.at[f, pl.ds(c * _OCH, _OCH)], sem)
                return 0
            lax.fori_loop(0, _B // (2 * _OCH), pair, 0, unroll=False)

        _L4 = 128

        jobs = []
        for tbl, out, phase in ((umlp, o_umlp, 0), (umf, o_umf, 0),
                                (uv, o_uv, 0), (uat, o_uat, 0),
                                (imlp, o_imlp, 1), (imf, o_imf, 1)):
            for kk in range(_CPT):
                jobs.append((tbl, out, kk, phase))

        pltpu.sync_copy(uidx_hbm, idx_v)
        csem = sem_c
        hs = pltpu.async_copy(jobs[0][0].at[wid * _CPT + jobs[0][2]],
                              col_v, csem)
        prev = None
        cur_phase = 0
        for t, (tbl, out, kk, phase) in enumerate(jobs):
            f = wid * _CPT + kk
            if phase != cur_phase:
                pltpu.sync_copy(iidx_hbm, idx_v)
                cur_phase = phase
            hs.wait()
            if prev is not None:
                pout, pf = prev
                for b in range(2):
                    ov, sem = obufs[b]
                    pltpu.make_async_copy(
                        ov, pout.at[pf, pl.ds(0, _OCH)], sem).wait()
            gather_column(tbl, out, f)
            if t + 1 < len(jobs):
                ntbl, _, nkk, _ = jobs[t + 1]
                hs = pltpu.async_copy(ntbl.at[wid * _CPT + nkk], col_v, csem)
            prev = (out, f)
        pout, pf = prev
        for b in range(2):
            ov, sem = obufs[b]
            pltpu.make_async_copy(ov, pout.at[pf, pl.ds(0, _OCH)], sem).wait()

    return k(uidx, iidx, tu_mlp, tu_mf, tu_v, tu_at, ti_mlp, ti_mf)


def _dgT(w, x):
    return lax.dot_general(w, x, (((0,), (0,)), ((), ())),
                           preferred_element_type=jnp.float32)


def _tc_poster_body(poster_ref, feW0_ref, feb0_ref, feW1_ref, feb1_ref,
                    pe_ref):
    f32 = jnp.float32
    h = jnp.maximum(
        lax.dot_general(feW0_ref[...], poster_ref[...],
                        (((0,), (1,)), ((), ())),
                        preferred_element_type=f32) + feb0_ref[...], 0.0)
    pe_ref[...] = _dgT(feW1_ref[...], h) + feb1_ref[...]


def _tc_poster(poster, feW0, feb0, feW1, feb1, bm=1024):
    return pl.pallas_call(
        _tc_poster_body,
        grid=(_B // bm,),
        in_specs=[
            pl.BlockSpec((bm, 2048), lambda i: (i, 0)),
            pl.BlockSpec(feW0.shape, lambda i: (0, 0)),
            pl.BlockSpec(feb0.shape, lambda i: (0, 0)),
            pl.BlockSpec(feW1.shape, lambda i: (0, 0)),
            pl.BlockSpec(feb1.shape, lambda i: (0, 0)),
        ],
        out_specs=pl.BlockSpec((_D, bm), lambda i: (0, i)),
        out_shape=jax.ShapeDtypeStruct((_D, _B), jnp.float32),
    )(poster, feW0, feb0, feW1, feb1)


def _tc_body(pe_ref, guml_ref, gumf_ref, guv_ref, guat_ref,
             giml_ref, gimf_ref,
             fcW0_ref, fcb0_ref, fcW1_ref, fcb1_ref,
             fvW0_ref, fvb0_ref, fvW1_ref, fvb1_ref,
             atW_ref, atb_ref, afW_ref, afb_ref, out_ref):
    f32 = jnp.float32
    pe_t = pe_ref[...]
    mlp_in = jnp.concatenate([guml_ref[...], giml_ref[...]], axis=0)
    mlp = jnp.maximum(_dgT(fcW0_ref[...], mlp_in) + fcb0_ref[...], 0.0)
    mlp = jnp.maximum(_dgT(fcW1_ref[...], mlp) + fcb1_ref[...], 0.0)

    v_in = jnp.concatenate([guv_ref[...], pe_t], axis=0)
    vv = jnp.maximum(_dgT(fvW0_ref[...], v_in) + fvb0_ref[...], 0.0)
    vv = jnp.maximum(_dgT(fvW1_ref[...], vv) + fvb1_ref[...], 0.0)

    att = jax.nn.sigmoid(
        _dgT(atW_ref[...], jnp.maximum(guat_ref[...], 0.0)) + atb_ref[...])

    mf = gumf_ref[...] * gimf_ref[...]
    vec = jnp.concatenate(
        [mlp * att[0:1, :], mf * att[1:2, :], vv * att[2:3, :]], axis=0)
    out_ref[...] = jax.nn.sigmoid(_dgT(afW_ref[...], vec) + afb_ref[...])


def _tc_dense(pe, guml, gumf, guv, guat, giml, gimf,
              fcW0, fcb0, fcW1, fcb1,
              fvW0, fvb0, fvW1, fvb1, atW, atb, afW, afb, bm=2048):
    grid = (_B // bm,)

    def col_spec(rows):
        return pl.BlockSpec((rows, bm), lambda i: (0, i))

    def full_spec(shape):
        return pl.BlockSpec(shape, lambda i: tuple(0 for _ in shape))

    in_specs = [
        col_spec(_D),
        col_spec(_D), col_spec(_D), col_spec(_D),
        col_spec(_D), col_spec(_D), col_spec(_D),
        full_spec(fcW0.shape), full_spec(fcb0.shape),
        full_spec(fcW1.shape), full_spec(fcb1.shape),
        full_spec(fvW0.shape), full_spec(fvb0.shape),
        full_spec(fvW1.shape), full_spec(fvb1.shape),
        full_spec(atW.shape), full_spec(atb.shape),
        full_spec(afW.shape), full_spec(afb.shape),
    ]
    return pl.pallas_call(
        _tc_body,
        grid=grid,
        in_specs=in_specs,
        out_specs=pl.BlockSpec((1, bm), lambda i: (0, i)),
        out_shape=jax.ShapeDtypeStruct((1, _B), jnp.float32),
    )(pe, guml, gumf, guv, guat, giml, gimf,
      fcW0, fcb0, fcW1, fcb1,
      fvW0, fvb0, fvW1, fvb1, atW, atb, afW, afb)


def kernel(user_indices, item_indices, poster_embeddings,
           emb_user_mlp, emb_item_mlp, emb_user_mf, emb_item_mf,
           emb_user_v, emb_atten,
           fe_W0, fe_b0, fe_W1, fe_b1, fc_W0, fc_b0, fc_W1, fc_b1,
           fv_W0, fv_b0, fv_W1, fv_b1, at_W, at_b, af_W, af_b):
    guml, gumf, guv, guat, giml, gimf = _sc_gather_cols(
        user_indices, item_indices,
        emb_user_mlp.T, emb_user_mf.T, emb_user_v.T, emb_atten.T,
        emb_item_mlp.T, emb_item_mf.T)
    pe = _tc_poster(poster_embeddings, fe_W0, fe_b0.reshape(-1, 1),
                    fe_W1, fe_b1.reshape(-1, 1))
    out_t = _tc_dense(
        pe, guml, gumf, guv, guat, giml, gimf,
        fc_W0, fc_b0.reshape(-1, 1), fc_W1, fc_b1.reshape(-1, 1),
        fv_W0, fv_b0.reshape(-1, 1), fv_W1, fv_b1.reshape(-1, 1),
        at_W, at_b.reshape(-1, 1), af_W, af_b.reshape(-1, 1))
    return out_t.reshape(_B, 1)
